# Initial kernel scaffold; baseline (speedup 1.0000x reference)
import jax
import jax.numpy as jnp
from jax.experimental import pallas as pl


def kernel(hidden_states, Wr, br, W1, b1, W2, b2, Wh, bh):
    raise NotImplementedError("write your pallas kernel here")


# speedup vs baseline: 1.3343x; 1.3343x over previous
import functools

import jax
import jax.numpy as jnp
from jax.experimental import pallas as pl
from jax.experimental.pallas import tpu as pltpu

EPS = 0.01
TOPK = 2
LAYERS = 2


def _halting(p, halting, rem, nu):
    thr = 1.0 - EPS
    still = (halting < 1.0).astype(p.dtype)
    hp = halting + p * still
    new_h = (hp > thr).astype(p.dtype) * still
    still = (hp <= thr).astype(p.dtype) * still
    halting = halting + p * still
    rem = rem + new_h * (1.0 - halting)
    halting = halting + new_h * rem
    nu = nu + still + new_h
    uw = p * still + new_h * rem
    return uw, halting, rem, nu


def _moe_act_kernel(x_ref, wr_ref, br_ref, w1_ref, b1_ref, w2_ref, b2_ref,
                    bh_ref, out_ref, sums_ref, *, n_experts, dff):
    e = pl.program_id(1)
    s = pl.program_id(0)
    xb = x_ref[...]
    T = xb.shape[0]

    logits = jnp.dot(xb, wr_ref[...], preferred_element_type=jnp.float32)
    logits = logits + br_ref[...]
    w = jax.nn.softmax(logits, axis=-1)
    iota = jax.lax.broadcasted_iota(jnp.int32, w.shape, 1)
    m1 = jnp.max(w, axis=1, keepdims=True)
    i1 = jnp.min(jnp.where(w == m1, iota, n_experts), axis=1, keepdims=True)
    wm = jnp.where(iota == i1, -jnp.inf, w)
    m2 = jnp.max(wm, axis=1, keepdims=True)
    i2 = jnp.min(jnp.where(wm == m2, iota, n_experts), axis=1, keepdims=True)
    we = jnp.sum(jnp.where(iota == e, w, 0.0), axis=1, keepdims=True)
    sel = (i1 == e) | (i2 == e)
    ew = jnp.where(sel, we / (m1 + m2), 0.0)

    w1 = w1_ref[0]
    w2 = w2_ref[0]
    bh = bh_ref[0, 0, 0]
    b1 = b1_ref[0]
    b2 = b2_ref[0]

    zero = jnp.zeros((T, 1), jnp.float32)
    halting, rem, nu = zero, zero, zero

    z1 = jnp.dot(xb, w1, preferred_element_type=jnp.float32) + b1
    p1 = jax.nn.sigmoid(z1[:, dff:dff + 1] + bh)
    uw1, halting, rem, nu = _halting(p1, halting, rem, nu)
    h1 = jax.nn.relu(z1[:, :dff])
    t1 = jnp.dot(h1, w2, preferred_element_type=jnp.float32) + b2

    z2 = jnp.dot(t1, w1, preferred_element_type=jnp.float32) + b1
    p2 = jax.nn.sigmoid(z2[:, dff:dff + 1] + bh)
    uw2, halting, rem, nu = _halting(p2, halting, rem, nu)
    h2 = jax.nn.relu(z2[:, :dff])
    t2 = jnp.dot(h2, w2, preferred_element_type=jnp.float32) + b2

    prev = t1 * uw1
    prev = t2 * uw2 + prev * (1.0 - uw2)
    contrib = prev * ew

    @pl.when(e == 0)
    def _():
        out_ref[...] = contrib

    @pl.when(e != 0)
    def _():
        out_ref[...] += contrib

    snr = jnp.sum(nu + rem)
    snu = jnp.sum(nu)
    sew = jnp.sum(ew)

    @pl.when(s == 0)
    def _():
        sums_ref[e, 0] = snr
        sums_ref[e, 1] = snu
        sums_ref[e, 2] = sew

    @pl.when(s != 0)
    def _():
        sums_ref[e, 0] += snr
        sums_ref[e, 1] += snu
        sums_ref[e, 2] += sew


@functools.partial(jax.jit, static_argnames=("interpret",))
def _run(hidden_states, Wr, br, W1, b1, W2, b2, Wh, bh, interpret=False):
    B, S, D = hidden_states.shape
    E = Wr.shape[1]
    DFF = W1.shape[2]
    T = 256
    ns = S // T

    x = hidden_states.reshape(S, D)
    whpad = jnp.pad(Wh, ((0, 0), (0, 0), (0, 127)))
    w1cat = jnp.concatenate([W1, whpad], axis=2)
    b1cat = jnp.pad(b1, ((0, 0), (0, 128)))
    DFC = DFF + 128

    grid = (ns, E)
    out, sums = pl.pallas_call(
        functools.partial(_moe_act_kernel, n_experts=E, dff=DFF),
        grid=grid,
        in_specs=[
            pl.BlockSpec((T, D), lambda s, e: (s, 0)),
            pl.BlockSpec((D, E), lambda s, e: (0, 0)),
            pl.BlockSpec((1, E), lambda s, e: (0, 0)),
            pl.BlockSpec((1, D, DFC), lambda s, e: (e, 0, 0)),
            pl.BlockSpec((1, 1, DFC), lambda s, e: (e, 0, 0)),
            pl.BlockSpec((1, DFF, D), lambda s, e: (e, 0, 0)),
            pl.BlockSpec((1, 1, D), lambda s, e: (e, 0, 0)),
            pl.BlockSpec((1, 1, 1), lambda s, e: (e, 0, 0)),
        ],
        out_specs=[
            pl.BlockSpec((T, D), lambda s, e: (s, 0)),
            pl.BlockSpec(memory_space=pltpu.SMEM),
        ],
        out_shape=[
            jax.ShapeDtypeStruct((S, D), jnp.float32),
            jax.ShapeDtypeStruct((E, 4), jnp.float32),
        ],
        interpret=interpret,
    )(x, Wr, br.reshape(1, E), w1cat, b1cat.reshape(E, 1, DFC), W2,
      b2.reshape(E, 1, D), bh.reshape(E, 1, 1))

    combined = out.reshape(B, S, D)
    n_tok = B * S
    el = sums[:, 0] / n_tok
    ep = sums[:, 1] / n_tok
    mew = sums[:, 2] / n_tok
    act_loss = jnp.sum(el * mew)
    ponder = jnp.sum(ep * mew)
    return combined, act_loss, ponder


def kernel(hidden_states, Wr, br, W1, b1, W2, b2, Wh, bh):
    return _run(hidden_states, Wr, br, W1, b1, W2, b2, Wh, bh)
